# baseline (speedup 1.0000x reference)
import functools

import jax
import jax.numpy as jnp
from jax import lax
from jax.experimental import pallas as pl
from jax.experimental.pallas import tpu as pltpu
from jax.experimental.pallas import tpu_sc as plsc

_B = 4096
_D = 64
_LANES = 16

_info = plsc.get_sparse_core_info()
_NC = _info.num_cores
_NS = _info.num_subcores
_NW = _NC * _NS
_BPW = _B // _NW

_mesh = plsc.VectorSubcoreMesh(core_axis_name="c", subcore_axis_name="s")


@functools.partial(
    pl.kernel,
    mesh=_mesh,
    compiler_params=pltpu.CompilerParams(needs_layout_passes=False),
    out_type=jax.ShapeDtypeStruct((_B,), jnp.float32),
    scratch_types=[
        pltpu.VMEM((_BPW,), jnp.int32),
        pltpu.VMEM((_BPW,), jnp.int32),
        pltpu.VMEM((_BPW, 2 * _D), jnp.float32),
        pltpu.VMEM((_BPW, 2 * _D), jnp.float32),
        pltpu.VMEM((_BPW,), jnp.float32),
        pltpu.SemaphoreType.DMA,
        pltpu.SemaphoreType.DMA,
    ],
)
def _gather_dot(users_hbm, items_hbm, tab_hbm, out_hbm,
                uidx, iidx, urows, irows, gout, sem_u, sem_i):
    wid = lax.axis_index("s") * _NC + lax.axis_index("c")
    base = wid * _BPW

    pltpu.sync_copy(users_hbm.at[pl.ds(base, _BPW)], uidx)
    pltpu.sync_copy(items_hbm.at[pl.ds(base, _BPW)], iidx)

    def fire(g, _):
        uvec = uidx[pl.ds(g * _LANES, _LANES)]
        ivec = iidx[pl.ds(g * _LANES, _LANES)]
        for r in range(_LANES):
            j = g * _LANES + r
            pltpu.async_copy(tab_hbm.at[pl.ds(uvec[r], 1), :],
                             urows.at[pl.ds(j, 1), :], sem_u)
            pltpu.async_copy(tab_hbm.at[pl.ds(ivec[r], 1), :],
                             irows.at[pl.ds(j, 1), :], sem_i)
        return 0

    lax.fori_loop(0, _BPW // _LANES, fire, 0)

    def drain(j, _):
        pltpu.make_async_copy(tab_hbm.at[pl.ds(0, 1), :],
                              urows.at[pl.ds(0, 1), :], sem_u).wait()
        pltpu.make_async_copy(tab_hbm.at[pl.ds(0, 1), :],
                              irows.at[pl.ds(0, 1), :], sem_i).wait()
        return 0

    lax.fori_loop(0, _BPW, drain, 0)

    lane = lax.iota(jnp.int32, _LANES)

    def group_body(g, _):
        acc = jnp.zeros((_LANES,), jnp.float32)
        for r in range(_LANES):
            b = g * _LANES + r
            s = urows[b, pl.ds(0, _LANES)] * irows[b, pl.ds(_D, _LANES)]
            for k in range(1, _D // _LANES):
                s = s + (urows[b, pl.ds(k * _LANES, _LANES)]
                         * irows[b, pl.ds(_D + k * _LANES, _LANES)])
            acc = jnp.where(lane == r, jnp.sum(s), acc)
        gout[pl.ds(g * _LANES, _LANES)] = acc
        return 0

    lax.fori_loop(0, _BPW // _LANES, group_body, 0)

    pltpu.sync_copy(gout, out_hbm.at[pl.ds(base, _BPW)])


def kernel(users, items, emb_user, emb_item, W1_w, W1_b, W2_w, W2_b,
           edge_index_g, vals_g, edge_index_gs, vals_gs):
    tab = jnp.concatenate([emb_user, emb_item], axis=1)
    return _gather_dot(users, items, tab)


# speedup vs baseline: 1.2842x; 1.2842x over previous
import functools

import jax
import jax.numpy as jnp
from jax import lax
from jax.experimental import pallas as pl
from jax.experimental.pallas import tpu as pltpu
from jax.experimental.pallas import tpu_sc as plsc

_B = 4096
_D = 64
_LANES = 16

_info = plsc.get_sparse_core_info()
_NC = _info.num_cores
_NS = _info.num_subcores
_NW = _NC * _NS
_BPW = _B // _NW

_mesh = plsc.VectorSubcoreMesh(core_axis_name="c", subcore_axis_name="s")


@functools.partial(
    pl.kernel,
    mesh=_mesh,
    compiler_params=pltpu.CompilerParams(needs_layout_passes=False),
    out_type=jax.ShapeDtypeStruct((_B,), jnp.float32),
    scratch_types=[
        pltpu.VMEM((_BPW,), jnp.int32),
        pltpu.VMEM((_BPW,), jnp.int32),
        pltpu.VMEM((_BPW, _D), jnp.float32),
        pltpu.VMEM((_BPW, _D), jnp.float32),
        pltpu.VMEM((_BPW,), jnp.float32),
        pltpu.SemaphoreType.DMA,
        pltpu.SemaphoreType.DMA,
    ],
)
def _gather_dot(users_hbm, items_hbm, eu_hbm, ei_hbm, out_hbm,
                uidx, iidx, urows, irows, gout, sem_u, sem_i):
    wid = lax.axis_index("s") * _NC + lax.axis_index("c")
    base = wid * _BPW

    pltpu.sync_copy(users_hbm.at[pl.ds(base, _BPW)], uidx)
    pltpu.sync_copy(items_hbm.at[pl.ds(base, _BPW)], iidx)

    def fire(g, _):
        uvec = uidx[pl.ds(g * _LANES, _LANES)]
        ivec = iidx[pl.ds(g * _LANES, _LANES)]
        for r in range(_LANES):
            j = g * _LANES + r
            pltpu.async_copy(eu_hbm.at[pl.ds(uvec[r], 1), :],
                             urows.at[pl.ds(j, 1), :], sem_u)
            pltpu.async_copy(ei_hbm.at[pl.ds(ivec[r], 1), :],
                             irows.at[pl.ds(j, 1), :], sem_i)
        return 0

    lax.fori_loop(0, _BPW // _LANES, fire, 0)

    pltpu.make_async_copy(eu_hbm.at[pl.ds(0, _BPW), :], urows, sem_u).wait()
    pltpu.make_async_copy(ei_hbm.at[pl.ds(0, _BPW), :], irows, sem_i).wait()

    lane = lax.iota(jnp.int32, _LANES)

    def group_body(g, _):
        acc = jnp.zeros((_LANES,), jnp.float32)
        for r in range(_LANES):
            b = g * _LANES + r
            s = urows[b, pl.ds(0, _LANES)] * irows[b, pl.ds(0, _LANES)]
            for k in range(1, _D // _LANES):
                s = s + (urows[b, pl.ds(k * _LANES, _LANES)]
                         * irows[b, pl.ds(k * _LANES, _LANES)])
            acc = jnp.where(lane == r, jnp.sum(s), acc)
        gout[pl.ds(g * _LANES, _LANES)] = acc
        return 0

    lax.fori_loop(0, _BPW // _LANES, group_body, 0)

    pltpu.sync_copy(gout, out_hbm.at[pl.ds(base, _BPW)])


def kernel(users, items, emb_user, emb_item, W1_w, W1_b, W2_w, W2_b,
           edge_index_g, vals_g, edge_index_gs, vals_gs):
    return _gather_dot(users, items, emb_user, emb_item)


# speedup vs baseline: 1.2954x; 1.0087x over previous
import functools

import jax
import jax.numpy as jnp
from jax import lax
from jax.experimental import pallas as pl
from jax.experimental.pallas import tpu as pltpu
from jax.experimental.pallas import tpu_sc as plsc

_B = 4096
_D = 64
_LANES = 16

_info = plsc.get_sparse_core_info()
_NC = _info.num_cores
_NS = _info.num_subcores
_NW = _NC * _NS
_BPW = _B // _NW

_mesh = plsc.VectorSubcoreMesh(core_axis_name="c", subcore_axis_name="s")


@functools.partial(
    pl.kernel,
    mesh=_mesh,
    compiler_params=pltpu.CompilerParams(needs_layout_passes=False),
    out_type=jax.ShapeDtypeStruct((_B,), jnp.float32),
    scratch_types=[
        pltpu.VMEM((_BPW,), jnp.int32),
        pltpu.VMEM((_BPW,), jnp.int32),
        pltpu.VMEM((_BPW, _D), jnp.float32),
        pltpu.VMEM((_BPW, _D), jnp.float32),
        pltpu.VMEM((_BPW,), jnp.float32),
        pltpu.SemaphoreType.DMA,
        pltpu.SemaphoreType.DMA,
    ],
)
def _gather_dot(users_hbm, items_hbm, eu_hbm, ei_hbm, out_hbm,
                uidx, iidx, urows, irows, gout, sem_u, sem_i):
    wid = lax.axis_index("s") * _NC + lax.axis_index("c")
    base = wid * _BPW

    pltpu.sync_copy(users_hbm.at[pl.ds(base, _BPW)], uidx)
    pltpu.sync_copy(items_hbm.at[pl.ds(base, _BPW)], iidx)

    def fire(g, _):
        uvec = uidx[pl.ds(g * _LANES, _LANES)]
        ivec = iidx[pl.ds(g * _LANES, _LANES)]
        for r in range(_LANES):
            j = g * _LANES + r
            pltpu.async_copy(eu_hbm.at[pl.ds(uvec[r], 1), :],
                             urows.at[pl.ds(j, 1), :], sem_u)
            pltpu.async_copy(ei_hbm.at[pl.ds(ivec[r], 1), :],
                             irows.at[pl.ds(j, 1), :], sem_i)
        return 0

    lax.fori_loop(0, _BPW // _LANES, fire, 0)

    pltpu.make_async_copy(eu_hbm.at[pl.ds(0, _BPW), :], urows, sem_u).wait()
    pltpu.make_async_copy(ei_hbm.at[pl.ds(0, _BPW), :], irows, sem_i).wait()

    lane = lax.iota(jnp.int32, _LANES)
    perms = [jnp.bitwise_xor(lane, step) for step in (1, 2, 4, 8)]

    def group_body(g, _):
        acc = jnp.zeros((_LANES,), jnp.float32)
        for r in range(_LANES):
            b = g * _LANES + r
            s = urows[b, pl.ds(0, _LANES)] * irows[b, pl.ds(0, _LANES)]
            for k in range(1, _D // _LANES):
                s = s + (urows[b, pl.ds(k * _LANES, _LANES)]
                         * irows[b, pl.ds(k * _LANES, _LANES)])
            for p in perms:
                s = s + s.at[p].get(mode="promise_in_bounds")
            acc = jnp.where(lane == r, s, acc)
        gout[pl.ds(g * _LANES, _LANES)] = acc
        return 0

    lax.fori_loop(0, _BPW // _LANES, group_body, 0)

    pltpu.sync_copy(gout, out_hbm.at[pl.ds(base, _BPW)])


def kernel(users, items, emb_user, emb_item, W1_w, W1_b, W2_w, W2_b,
           edge_index_g, vals_g, edge_index_gs, vals_gs):
    return _gather_dot(users, items, emb_user, emb_item)
